# baseline (speedup 1.0000x reference)
import jax
import jax.numpy as jnp
from jax.experimental import pallas as pl
from jax.experimental.pallas import tpu as pltpu

B, S, D, H = 2, 2048, 2048, 16
HD = D // H
R = int(S * 0.9)

BM = 512
KB = 512


def _attn_body(q_ref, k_ref, v_ref, o_ref, imp_ref, acc_ref):
    j = pl.program_id(1)

    @pl.when(j == 0)
    def _init():
        acc_ref[...] = jnp.zeros_like(acc_ref)

    q = q_ref[0, 0]
    k = k_ref[0, 0]
    v = v_ref[0]
    s = jnp.dot(q, k.T, preferred_element_type=jnp.float32) / jnp.sqrt(
        jnp.float32(HD))
    m = jnp.max(s, axis=0, keepdims=True)
    e = jnp.exp(s - m)
    z = jnp.sum(e, axis=0, keepdims=True)
    a = e / z
    imp_ref[0, 0] = jnp.sum(a, axis=0, keepdims=True) / jnp.float32(S)
    acc_ref[...] += jnp.dot(a.astype(jnp.bfloat16), v,
                            preferred_element_type=jnp.float32)

    @pl.when(j == pl.num_programs(1) - 1)
    def _done():
        o_ref[0] = acc_ref[...]


def _attention(qp, kp, vb):
    grid = (B * H, S // KB)
    return pl.pallas_call(
        _attn_body,
        grid=grid,
        in_specs=[
            pl.BlockSpec((1, 1, S, HD), lambda bh, j: (bh // H, bh % H, 0, 0)),
            pl.BlockSpec((1, 1, KB, HD), lambda bh, j: (bh // H, bh % H, j, 0)),
            pl.BlockSpec((1, KB, HD), lambda bh, j: (bh // H, j, bh % H)),
        ],
        out_specs=[
            pl.BlockSpec((1, S, HD), lambda bh, j: (bh // H, 0, bh % H)),
            pl.BlockSpec((1, 1, 1, KB), lambda bh, j: (bh // H, bh % H, 0, j)),
        ],
        out_shape=[
            jax.ShapeDtypeStruct((B, S, D), jnp.float32),
            jax.ShapeDtypeStruct((B, H, 1, S), jnp.float32),
        ],
        scratch_shapes=[pltpu.VMEM((S, HD), jnp.float32)],
        compiler_params=pltpu.CompilerParams(
            dimension_semantics=("arbitrary", "arbitrary"),
        ),
    )(qp, kp, vb)


def _mm_body(x_ref, w_ref, b_ref, o_ref):
    acc = jnp.dot(x_ref[...], w_ref[...], preferred_element_type=jnp.float32)
    o_ref[0] = (acc + b_ref[0]).astype(o_ref.dtype)


BN = 512


def _mm(x2d, wT, b3d, out_dtype):
    grid = (B * S // BM, D // BN)
    return pl.pallas_call(
        _mm_body,
        grid=grid,
        in_specs=[
            pl.BlockSpec((BM, D), lambda i, j: (i, 0)),
            pl.BlockSpec((D, BN), lambda i, j: (0, j)),
            pl.BlockSpec((1, 1, BN), lambda i, j: (j, 0, 0)),
        ],
        out_specs=pl.BlockSpec(
            (1, BM, BN), lambda i, j: (i // (S // BM), i % (S // BM), j)
        ),
        out_shape=jax.ShapeDtypeStruct((B, S, D), out_dtype),
        compiler_params=pltpu.CompilerParams(
            dimension_semantics=("parallel", "parallel"),
        ),
    )(x2d, wT, b3d)


def kernel(x, Wq, bq, Wk, bk, Wv, bv, Wo, bo):
    Q = x @ Wq.T + bq
    K = x @ Wk.T + bk
    qp = Q.reshape(B, S, H, HD).transpose(0, 2, 1, 3)
    kp = K.reshape(B, S, H, HD).transpose(0, 2, 1, 3)

    x2d = x.reshape(B * S, D).astype(jnp.bfloat16)
    vb = _mm(x2d, Wv.T.astype(jnp.bfloat16), bv.reshape(D // BN, 1, BN),
             jnp.bfloat16)
    attn_out, importance4 = _attention(qp, kp, vb)
    topk_indices = jax.lax.top_k(importance4[:, :, 0, :], R)[1]
    y2d = attn_out.reshape(B * S, D).astype(jnp.bfloat16)
    out_p = _mm(y2d, Wo.T.astype(jnp.bfloat16), bo.reshape(D // BN, 1, BN),
                jnp.float32)
    return (out_p, topk_indices)


# speedup vs baseline: 1.1083x; 1.1083x over previous
import jax
import jax.numpy as jnp
from jax.experimental import pallas as pl
from jax.experimental.pallas import tpu as pltpu

B, S, D, H = 2, 2048, 2048, 16
HD = D // H
R = int(S * 0.9)

BM = 512
KB = 1024


def _attn_body(q_ref, k_ref, v_ref, o_ref, imp_ref, acc_ref):
    j = pl.program_id(1)

    @pl.when(j == 0)
    def _init():
        acc_ref[...] = jnp.zeros_like(acc_ref)

    q = q_ref[0, 0]
    k = k_ref[0, 0]
    v = v_ref[0]
    s = jnp.dot(q, k.T, preferred_element_type=jnp.float32) / jnp.sqrt(
        jnp.float32(HD))
    m = jnp.max(s, axis=0, keepdims=True)
    e = jnp.exp(s - m)
    z = jnp.sum(e, axis=0, keepdims=True)
    a = e * (1.0 / z)
    imp_ref[0, 0] = jnp.sum(a, axis=0, keepdims=True) / jnp.float32(S)
    acc_ref[...] += jnp.dot(a.astype(jnp.bfloat16), v,
                            preferred_element_type=jnp.float32)

    @pl.when(j == pl.num_programs(1) - 1)
    def _done():
        o_ref[0] = acc_ref[...]


def _attention(qp, kp, vb):
    grid = (B * H, S // KB)
    return pl.pallas_call(
        _attn_body,
        grid=grid,
        in_specs=[
            pl.BlockSpec((1, 1, S, HD), lambda bh, j: (bh // H, bh % H, 0, 0)),
            pl.BlockSpec((1, 1, KB, HD), lambda bh, j: (bh // H, bh % H, j, 0)),
            pl.BlockSpec((1, KB, HD), lambda bh, j: (bh // H, j, bh % H)),
        ],
        out_specs=[
            pl.BlockSpec((1, S, HD), lambda bh, j: (bh // H, 0, bh % H)),
            pl.BlockSpec((1, 1, 1, KB), lambda bh, j: (bh // H, bh % H, 0, j)),
        ],
        out_shape=[
            jax.ShapeDtypeStruct((B, S, D), jnp.float32),
            jax.ShapeDtypeStruct((B, H, 1, S), jnp.float32),
        ],
        scratch_shapes=[pltpu.VMEM((S, HD), jnp.float32)],
        compiler_params=pltpu.CompilerParams(
            dimension_semantics=("arbitrary", "arbitrary"),
        ),
    )(qp, kp, vb)


def _mm_body(x_ref, w_ref, b_ref, o_ref):
    acc = jnp.dot(x_ref[...], w_ref[...], preferred_element_type=jnp.float32)
    o_ref[0] = (acc + b_ref[0]).astype(o_ref.dtype)


BN = 512


def _mm(x2d, wT, b3d, out_dtype):
    grid = (B * S // BM, D // BN)
    return pl.pallas_call(
        _mm_body,
        grid=grid,
        in_specs=[
            pl.BlockSpec((BM, D), lambda i, j: (i, 0)),
            pl.BlockSpec((D, BN), lambda i, j: (0, j)),
            pl.BlockSpec((1, 1, BN), lambda i, j: (j, 0, 0)),
        ],
        out_specs=pl.BlockSpec(
            (1, BM, BN), lambda i, j: (i // (S // BM), i % (S // BM), j)
        ),
        out_shape=jax.ShapeDtypeStruct((B, S, D), out_dtype),
        compiler_params=pltpu.CompilerParams(
            dimension_semantics=("parallel", "parallel"),
        ),
    )(x2d, wT, b3d)


def kernel(x, Wq, bq, Wk, bk, Wv, bv, Wo, bo):
    Q = x @ Wq.T + bq
    K = x @ Wk.T + bk
    qp = Q.reshape(B, S, H, HD).transpose(0, 2, 1, 3)
    kp = K.reshape(B, S, H, HD).transpose(0, 2, 1, 3)

    x2d = x.reshape(B * S, D).astype(jnp.bfloat16)
    vb = _mm(x2d, Wv.T.astype(jnp.bfloat16), bv.reshape(D // BN, 1, BN),
             jnp.bfloat16)
    attn_out, importance4 = _attention(qp, kp, vb)
    topk_indices = jax.lax.top_k(importance4[:, :, 0, :], R)[1]
    y2d = attn_out.reshape(B * S, D).astype(jnp.bfloat16)
    out_p = _mm(y2d, Wo.T.astype(jnp.bfloat16), bo.reshape(D // BN, 1, BN),
                jnp.float32)
    return (out_p, topk_indices)


# speedup vs baseline: 1.1093x; 1.0009x over previous
import jax
import jax.numpy as jnp
from jax.experimental import pallas as pl
from jax.experimental.pallas import tpu as pltpu

B, S, D, H = 2, 2048, 2048, 16
HD = D // H
R = int(S * 0.9)

BM = 512
KB = 1024


def _attn_body(q_ref, k_ref, v_ref, o_ref, imp_ref, acc_ref):
    j = pl.program_id(1)

    @pl.when(j == 0)
    def _init():
        acc_ref[...] = jnp.zeros_like(acc_ref)

    q = q_ref[0, 0]
    k = k_ref[0, 0]
    v = v_ref[0]
    s = jnp.dot(q, k.T, preferred_element_type=jnp.float32) / jnp.sqrt(
        jnp.float32(HD))
    m = jnp.max(s, axis=0, keepdims=True)
    e = jnp.exp(s - m)
    z = jnp.sum(e, axis=0, keepdims=True)
    a = e * (1.0 / z)
    imp_ref[0, 0] = jnp.sum(a, axis=0, keepdims=True) / jnp.float32(S)
    acc_ref[...] += jnp.dot(a.astype(jnp.bfloat16), v,
                            preferred_element_type=jnp.float32)

    @pl.when(j == pl.num_programs(1) - 1)
    def _done():
        o_ref[0] = acc_ref[...]


def _attention(qp, kp, vb):
    grid = (B * H, S // KB)
    return pl.pallas_call(
        _attn_body,
        grid=grid,
        in_specs=[
            pl.BlockSpec((1, 1, S, HD), lambda bh, j: (bh // H, bh % H, 0, 0)),
            pl.BlockSpec((1, 1, KB, HD), lambda bh, j: (bh // H, bh % H, j, 0)),
            pl.BlockSpec((1, KB, HD), lambda bh, j: (bh // H, j, bh % H)),
        ],
        out_specs=[
            pl.BlockSpec((1, S, HD), lambda bh, j: (bh // H, 0, bh % H)),
            pl.BlockSpec((1, 1, 1, KB), lambda bh, j: (bh // H, bh % H, 0, j)),
        ],
        out_shape=[
            jax.ShapeDtypeStruct((B, S, D), jnp.float32),
            jax.ShapeDtypeStruct((B, H, 1, S), jnp.float32),
        ],
        scratch_shapes=[pltpu.VMEM((S, HD), jnp.float32)],
        compiler_params=pltpu.CompilerParams(
            dimension_semantics=("parallel", "arbitrary"),
        ),
    )(qp, kp, vb)


def _mm_body(x_ref, w_ref, b_ref, o_ref):
    acc = jnp.dot(x_ref[...], w_ref[...], preferred_element_type=jnp.float32)
    o_ref[0] = (acc + b_ref[0]).astype(o_ref.dtype)


BN = 512


def _mm(x2d, wT, b3d, out_dtype):
    grid = (B * S // BM, D // BN)
    return pl.pallas_call(
        _mm_body,
        grid=grid,
        in_specs=[
            pl.BlockSpec((BM, D), lambda i, j: (i, 0)),
            pl.BlockSpec((D, BN), lambda i, j: (0, j)),
            pl.BlockSpec((1, 1, BN), lambda i, j: (j, 0, 0)),
        ],
        out_specs=pl.BlockSpec(
            (1, BM, BN), lambda i, j: (i // (S // BM), i % (S // BM), j)
        ),
        out_shape=jax.ShapeDtypeStruct((B, S, D), out_dtype),
        compiler_params=pltpu.CompilerParams(
            dimension_semantics=("parallel", "parallel"),
        ),
    )(x2d, wT, b3d)


def kernel(x, Wq, bq, Wk, bk, Wv, bv, Wo, bo):
    Q = x @ Wq.T + bq
    K = x @ Wk.T + bk
    qp = Q.reshape(B, S, H, HD).transpose(0, 2, 1, 3)
    kp = K.reshape(B, S, H, HD).transpose(0, 2, 1, 3)

    x2d = x.reshape(B * S, D).astype(jnp.bfloat16)
    vb = _mm(x2d, Wv.T.astype(jnp.bfloat16), bv.reshape(D // BN, 1, BN),
             jnp.bfloat16)
    attn_out, importance4 = _attention(qp, kp, vb)
    topk_indices = jax.lax.top_k(importance4[:, :, 0, :], R)[1]
    y2d = attn_out.reshape(B * S, D).astype(jnp.bfloat16)
    out_p = _mm(y2d, Wo.T.astype(jnp.bfloat16), bo.reshape(D // BN, 1, BN),
                jnp.float32)
    return (out_p, topk_indices)


# speedup vs baseline: 1.1591x; 1.0449x over previous
import jax
import jax.numpy as jnp
from jax.experimental import pallas as pl
from jax.experimental.pallas import tpu as pltpu

B, S, D, H = 2, 2048, 2048, 16
HD = D // H
R = int(S * 0.9)

BM = 1024
KB = 1024


def _attn_body(q_ref, k_ref, v_ref, o_ref, imp_ref, acc_ref):
    j = pl.program_id(1)

    @pl.when(j == 0)
    def _init():
        acc_ref[...] = jnp.zeros_like(acc_ref)

    q = q_ref[0, 0]
    k = k_ref[0, 0]
    v = v_ref[0]
    s = jnp.dot(q, k.T, preferred_element_type=jnp.float32) / jnp.sqrt(
        jnp.float32(HD))
    m = jnp.max(s, axis=0, keepdims=True)
    e = jnp.exp(s - m)
    z = jnp.sum(e, axis=0, keepdims=True)
    zr = 1.0 / z
    imp_ref[0, 0] = jnp.sum(e * zr, axis=0, keepdims=True) / jnp.float32(S)
    acc_ref[...] += jnp.dot((e * zr).astype(jnp.bfloat16), v,
                            preferred_element_type=jnp.float32)

    @pl.when(j == pl.num_programs(1) - 1)
    def _done():
        o_ref[0] = acc_ref[...]


def _attention(qp, kp, vb):
    grid = (B * H, S // KB)
    return pl.pallas_call(
        _attn_body,
        grid=grid,
        in_specs=[
            pl.BlockSpec((1, 1, S, HD), lambda bh, j: (bh // H, bh % H, 0, 0)),
            pl.BlockSpec((1, 1, KB, HD), lambda bh, j: (bh // H, bh % H, j, 0)),
            pl.BlockSpec((1, KB, HD), lambda bh, j: (bh // H, j, bh % H)),
        ],
        out_specs=[
            pl.BlockSpec((1, S, HD), lambda bh, j: (bh // H, 0, bh % H)),
            pl.BlockSpec((1, 1, 1, KB), lambda bh, j: (bh // H, bh % H, 0, j)),
        ],
        out_shape=[
            jax.ShapeDtypeStruct((B, S, D), jnp.float32),
            jax.ShapeDtypeStruct((B, H, 1, S), jnp.float32),
        ],
        scratch_shapes=[pltpu.VMEM((S, HD), jnp.float32)],
        compiler_params=pltpu.CompilerParams(
            dimension_semantics=("parallel", "arbitrary"),
        ),
    )(qp, kp, vb)


def _mm_body(x_ref, w_ref, b_ref, o_ref):
    acc = jnp.dot(x_ref[...], w_ref[...], preferred_element_type=jnp.float32)
    o_ref[0] = (acc + b_ref[0]).astype(o_ref.dtype)


BN = 512


def _mm(x2d, wT, b3d, out_dtype):
    grid = (B * S // BM, D // BN)
    return pl.pallas_call(
        _mm_body,
        grid=grid,
        in_specs=[
            pl.BlockSpec((BM, D), lambda i, j: (i, 0)),
            pl.BlockSpec((D, BN), lambda i, j: (0, j)),
            pl.BlockSpec((1, 1, BN), lambda i, j: (j, 0, 0)),
        ],
        out_specs=pl.BlockSpec(
            (1, BM, BN), lambda i, j: (i // (S // BM), i % (S // BM), j)
        ),
        out_shape=jax.ShapeDtypeStruct((B, S, D), out_dtype),
        compiler_params=pltpu.CompilerParams(
            dimension_semantics=("parallel", "parallel"),
        ),
    )(x2d, wT, b3d)


def kernel(x, Wq, bq, Wk, bk, Wv, bv, Wo, bo):
    Q = x @ Wq.T + bq
    K = x @ Wk.T + bk
    qp = Q.reshape(B, S, H, HD).transpose(0, 2, 1, 3)
    kp = K.reshape(B, S, H, HD).transpose(0, 2, 1, 3)

    x2d = x.reshape(B * S, D).astype(jnp.bfloat16)
    vb = _mm(x2d, Wv.T.astype(jnp.bfloat16), bv.reshape(D // BN, 1, BN),
             jnp.bfloat16)
    attn_out, importance4 = _attention(qp, kp, vb)
    topk_indices = jax.lax.top_k(importance4[:, :, 0, :], R)[1]
    y2d = attn_out.reshape(B * S, D).astype(jnp.bfloat16)
    out_p = _mm(y2d, Wo.T.astype(jnp.bfloat16), bo.reshape(D // BN, 1, BN),
                jnp.float32)
    return (out_p, topk_indices)


# speedup vs baseline: 1.1920x; 1.0283x over previous
import jax
import jax.numpy as jnp
from jax.experimental import pallas as pl
from jax.experimental.pallas import tpu as pltpu

B, S, D, H = 2, 2048, 2048, 16
HD = D // H
R = int(S * 0.9)

BM = 1024
KB = 1024


def _attn_body(q_ref, k_ref, v_ref, o_ref, imp_ref, acc_ref):
    j = pl.program_id(1)

    @pl.when(j == 0)
    def _init():
        acc_ref[...] = jnp.zeros_like(acc_ref)

    q = q_ref[0, 0]
    k = k_ref[0, 0]
    v = v_ref[0]
    s = jnp.dot(q, k.T, preferred_element_type=jnp.float32) / jnp.sqrt(
        jnp.float32(HD))
    m = jnp.max(s, axis=0, keepdims=True)
    e = jnp.exp(s - m)
    z = jnp.sum(e, axis=0, keepdims=True)
    zr = 1.0 / z
    imp_ref[0, 0] = jnp.sum(e * zr, axis=0, keepdims=True) / jnp.float32(S)
    acc_ref[...] += jnp.dot((e * zr).astype(jnp.bfloat16), v,
                            preferred_element_type=jnp.float32)

    @pl.when(j == pl.num_programs(1) - 1)
    def _done():
        o_ref[0] = acc_ref[...].astype(jnp.bfloat16)


def _attention(qp, kp, vb):
    grid = (B * H, S // KB)
    return pl.pallas_call(
        _attn_body,
        grid=grid,
        in_specs=[
            pl.BlockSpec((1, 1, S, HD), lambda bh, j: (bh // H, bh % H, 0, 0)),
            pl.BlockSpec((1, 1, KB, HD), lambda bh, j: (bh // H, bh % H, j, 0)),
            pl.BlockSpec((1, KB, HD), lambda bh, j: (bh // H, j, bh % H)),
        ],
        out_specs=[
            pl.BlockSpec((1, S, HD), lambda bh, j: (bh // H, 0, bh % H)),
            pl.BlockSpec((1, 1, 1, KB), lambda bh, j: (bh // H, bh % H, 0, j)),
        ],
        out_shape=[
            jax.ShapeDtypeStruct((B, S, D), jnp.bfloat16),
            jax.ShapeDtypeStruct((B, H, 1, S), jnp.float32),
        ],
        scratch_shapes=[pltpu.VMEM((S, HD), jnp.float32)],
        compiler_params=pltpu.CompilerParams(
            dimension_semantics=("parallel", "arbitrary"),
        ),
    )(qp, kp, vb)


def _mm_body(x_ref, w_ref, b_ref, o_ref):
    acc = jnp.dot(x_ref[...], w_ref[...], preferred_element_type=jnp.float32)
    o_ref[0] = (acc + b_ref[0]).astype(o_ref.dtype)


BN = 512


def _mm(x2d, wT, b3d, out_dtype):
    grid = (B * S // BM, D // BN)
    return pl.pallas_call(
        _mm_body,
        grid=grid,
        in_specs=[
            pl.BlockSpec((BM, D), lambda i, j: (i, 0)),
            pl.BlockSpec((D, BN), lambda i, j: (0, j)),
            pl.BlockSpec((1, 1, BN), lambda i, j: (j, 0, 0)),
        ],
        out_specs=pl.BlockSpec(
            (1, BM, BN), lambda i, j: (i // (S // BM), i % (S // BM), j)
        ),
        out_shape=jax.ShapeDtypeStruct((B, S, D), out_dtype),
        compiler_params=pltpu.CompilerParams(
            dimension_semantics=("parallel", "parallel"),
        ),
    )(x2d, wT, b3d)


def kernel(x, Wq, bq, Wk, bk, Wv, bv, Wo, bo):
    Q = x @ Wq.T + bq
    K = x @ Wk.T + bk
    qp = Q.reshape(B, S, H, HD).transpose(0, 2, 1, 3)
    kp = K.reshape(B, S, H, HD).transpose(0, 2, 1, 3)

    x2d = x.reshape(B * S, D).astype(jnp.bfloat16)
    vb = _mm(x2d, Wv.T.astype(jnp.bfloat16), bv.reshape(D // BN, 1, BN),
             jnp.bfloat16)
    attn_out, importance4 = _attention(qp, kp, vb)
    topk_indices = jax.lax.top_k(importance4[:, :, 0, :], R)[1]
    y2d = attn_out.reshape(B * S, D)
    out_p = _mm(y2d, Wo.T.astype(jnp.bfloat16), bo.reshape(D // BN, 1, BN),
                jnp.float32)
    return (out_p, topk_indices)


# speedup vs baseline: 1.2728x; 1.0678x over previous
import jax
import jax.numpy as jnp
from jax.experimental import pallas as pl
from jax.experimental.pallas import tpu as pltpu

B, S, D, H = 2, 2048, 2048, 16
HD = D // H
R = int(S * 0.9)

BM = 1024
KB = 1024


def _attn_body(q_ref, k_ref, v_ref, o_ref, imp_ref, acc_ref):
    j = pl.program_id(1)

    @pl.when(j == 0)
    def _init():
        acc_ref[...] = jnp.zeros_like(acc_ref)

    q = q_ref[0, 0]
    k = k_ref[0, 0]
    v = v_ref[0]
    s = jnp.dot(q, k.T, preferred_element_type=jnp.float32) / jnp.sqrt(
        jnp.float32(HD))
    m = jnp.max(s, axis=0, keepdims=True)
    e = jnp.exp(s - m)
    z = jnp.sum(e, axis=0, keepdims=True)
    zr = 1.0 / z
    imp_ref[0, 0] = jnp.sum(e * zr, axis=0, keepdims=True) / jnp.float32(S)
    acc_ref[...] += jnp.dot((e * zr).astype(jnp.bfloat16), v,
                            preferred_element_type=jnp.float32)

    @pl.when(j == pl.num_programs(1) - 1)
    def _done():
        o_ref[0] = acc_ref[...].astype(jnp.bfloat16)


def _attention(qp, kp, vb):
    grid = (B * H, S // KB)
    return pl.pallas_call(
        _attn_body,
        grid=grid,
        in_specs=[
            pl.BlockSpec((1, 1, S, HD), lambda bh, j: (bh // H, bh % H, 0, 0)),
            pl.BlockSpec((1, 1, KB, HD), lambda bh, j: (bh // H, bh % H, j, 0)),
            pl.BlockSpec((1, KB, HD), lambda bh, j: (bh // H, j, bh % H)),
        ],
        out_specs=[
            pl.BlockSpec((1, S, HD), lambda bh, j: (bh // H, 0, bh % H)),
            pl.BlockSpec((1, 1, 1, KB), lambda bh, j: (bh // H, bh % H, 0, j)),
        ],
        out_shape=[
            jax.ShapeDtypeStruct((B, S, D), jnp.bfloat16),
            jax.ShapeDtypeStruct((B, H, 1, S), jnp.float32),
        ],
        scratch_shapes=[pltpu.VMEM((S, HD), jnp.float32)],
        compiler_params=pltpu.CompilerParams(
            dimension_semantics=("parallel", "arbitrary"),
        ),
    )(qp, kp, vb)


def _mm_body(x_ref, w_ref, b_ref, o_ref):
    acc = jnp.dot(x_ref[...], w_ref[...], preferred_element_type=jnp.float32)
    o_ref[0] = (acc + b_ref[0]).astype(o_ref.dtype)


BN = 512


def _mm(x2d, wT, b3d, out_dtype):
    grid = (B * S // BM, D // BN)
    return pl.pallas_call(
        _mm_body,
        grid=grid,
        in_specs=[
            pl.BlockSpec((BM, D), lambda i, j: (i, 0)),
            pl.BlockSpec((D, BN), lambda i, j: (0, j)),
            pl.BlockSpec((1, 1, BN), lambda i, j: (j, 0, 0)),
        ],
        out_specs=pl.BlockSpec(
            (1, BM, BN), lambda i, j: (i // (S // BM), i % (S // BM), j)
        ),
        out_shape=jax.ShapeDtypeStruct((B, S, D), out_dtype),
        compiler_params=pltpu.CompilerParams(
            dimension_semantics=("parallel", "parallel"),
        ),
    )(x2d, wT, b3d)


def kernel(x, Wq, bq, Wk, bk, Wv, bv, Wo, bo):
    Q = x @ Wq.T + bq
    K = x @ Wk.T + bk
    qp = Q.reshape(B, S, H, HD).transpose(0, 2, 1, 3)
    kp = K.reshape(B, S, H, HD).transpose(0, 2, 1, 3)

    vb = (x @ Wv.T + bv).astype(jnp.bfloat16)
    attn_out, importance4 = _attention(qp, kp, vb)
    topk_indices = jax.lax.top_k(importance4[:, :, 0, :], R)[1]
    y2d = attn_out.reshape(B * S, D)
    out_p = _mm(y2d, Wo.T.astype(jnp.bfloat16), bo.reshape(D // BN, 1, BN),
                jnp.float32)
    return (out_p, topk_indices)
